# Initial kernel scaffold; baseline (speedup 1.0000x reference)
import jax
import jax.numpy as jnp
from jax.experimental import pallas as pl


def kernel(node_features, node_attributes, sph_harmonics, radial_basis, edge_index, W_pre, W_mlp1, W_mlp2, W_mlp3, W_post_int, W_contr, W_sc, W_post):
    raise NotImplementedError("write your pallas kernel here")


# speedup vs baseline: 1.7153x; 1.7153x over previous
import functools

import jax
import jax.numpy as jnp
from jax import lax
from jax.experimental import pallas as pl
from jax.experimental.pallas import tpu as pltpu
from jax.experimental.pallas import tpu_sc as plsc

N = 10000
E = 320000
C = 128
A = 10
RB = 8
H = 64
CORR = 3

NC = 2
NS = 16
NW = NC * NS
K = 128
NCHUNK = 79
EPW = K * NCHUNK
E_PAD = EPW * NW
ROWS_PER_TILE = 632
N_PAD = ROWS_PER_TILE * NS

_BLK_N = 1000
_BLK_E = 2048


def _pre_body(nf_ref, w_ref, h_ref):
    h_ref[...] = jnp.dot(nf_ref[...], w_ref[...],
                         preferred_element_type=jnp.float32)


def _scale_body(rb_ref, sph_ref, w1_ref, w2_ref, w3_ref, s_ref):
    x = jax.nn.silu(jnp.dot(rb_ref[...], w1_ref[...],
                            preferred_element_type=jnp.float32))
    x = jax.nn.silu(jnp.dot(x, w2_ref[...],
                            preferred_element_type=jnp.float32))
    w = jnp.dot(x, w3_ref[...], preferred_element_type=jnp.float32)
    s_ref[...] = w * sph_ref[...]


def _post_body(p_ref, nf_ref, attr_ref, wpi_ref, wc_ref, wsct_ref, wp_ref,
               out_ref):
    tm = p_ref[0] + p_ref[1]
    m = jnp.dot(tm, wpi_ref[...], preferred_element_type=jnp.float32)
    attr = attr_ref[...]
    m2 = m * m
    contracted = (jnp.dot(attr, wc_ref[0], preferred_element_type=jnp.float32) * m
                  + jnp.dot(attr, wc_ref[1], preferred_element_type=jnp.float32) * m2
                  + jnp.dot(attr, wc_ref[2], preferred_element_type=jnp.float32) * (m2 * m))
    nf = nf_ref[...]
    sc = jnp.zeros_like(m)
    for a in range(A):
        sc = sc + attr[:, a:a + 1] * jnp.dot(nf, wsct_ref[a],
                                             preferred_element_type=jnp.float32)
    out_ref[...] = jnp.dot(contracted + sc, wp_ref[...],
                           preferred_element_type=jnp.float32)


def _agg_body(h_hbm, s_hbm, src_hbm, dst_hbm, zeros_hbm, out_hbm,
              src_v, dst_v, rows_v, s_v, acc, sem):
    cid = lax.axis_index("c")
    sid = lax.axis_index("s")
    wid = sid * NC + cid
    base = wid * EPW
    my_rows = pl.ds(sid * ROWS_PER_TILE, ROWS_PER_TILE)

    pltpu.sync_copy(zeros_hbm.at[my_rows], acc.at[my_rows])
    plsc.subcore_barrier()

    def mul_row(r, carry):
        for gi in range(C // 16):
            sl = pl.ds(gi * 16, 16)
            rows_v[r, sl] = rows_v[r, sl] * s_v[r, sl]
        return carry

    def chunk_body(i, carry):
        off = base + i * K
        pltpu.sync_copy(src_hbm.at[pl.ds(off, K)], src_v)
        pltpu.sync_copy(dst_hbm.at[pl.ds(off, K)], dst_v)
        pltpu.async_copy(h_hbm.at[src_v], rows_v, sem).wait()
        pltpu.sync_copy(s_hbm.at[pl.ds(off, K)], s_v)
        lax.fori_loop(0, K, mul_row, 0)
        pltpu.sync_copy(rows_v, acc.at[dst_v], add=True)
        return carry

    lax.fori_loop(0, NCHUNK, chunk_body, 0)

    plsc.subcore_barrier()
    pltpu.sync_copy(acc.at[my_rows], out_hbm.at[cid].at[my_rows])


_agg_kernel = functools.partial(
    pl.kernel,
    out_type=jax.ShapeDtypeStruct((NC, N_PAD, C), jnp.float32),
    mesh=plsc.VectorSubcoreMesh(core_axis_name="c", subcore_axis_name="s"),
    scratch_types=[
        pltpu.VMEM((K,), jnp.int32),
        pltpu.VMEM((K,), jnp.int32),
        pltpu.VMEM((K, C), jnp.float32),
        pltpu.VMEM((K, C), jnp.float32),
        pltpu.VMEM_SHARED((N_PAD, C), jnp.float32),
        pltpu.SemaphoreType.DMA,
    ],
)(_agg_body)


def kernel(node_features, node_attributes, sph_harmonics, radial_basis,
           edge_index, W_pre, W_mlp1, W_mlp2, W_mlp3, W_post_int,
           W_contr, W_sc, W_post):
    f32 = jnp.float32
    pad = E_PAD - E
    src = jnp.concatenate([edge_index[0], jnp.zeros((pad,), jnp.int32)])
    dst = jnp.concatenate([edge_index[1], jnp.full((pad,), N, jnp.int32)])
    rb_p = jnp.concatenate([radial_basis, jnp.zeros((pad, RB), f32)])
    sph_p = jnp.concatenate([sph_harmonics, jnp.zeros((pad, 1), f32)])
    zeros = jnp.zeros((N_PAD, C), f32)

    h = pl.pallas_call(
        _pre_body,
        grid=(N // _BLK_N,),
        in_specs=[pl.BlockSpec((_BLK_N, C), lambda i: (i, 0)),
                  pl.BlockSpec((C, C), lambda i: (0, 0))],
        out_specs=pl.BlockSpec((_BLK_N, C), lambda i: (i, 0)),
        out_shape=jax.ShapeDtypeStruct((N, C), f32),
    )(node_features, W_pre)

    s = pl.pallas_call(
        _scale_body,
        grid=(E_PAD // _BLK_E,),
        in_specs=[pl.BlockSpec((_BLK_E, RB), lambda i: (i, 0)),
                  pl.BlockSpec((_BLK_E, 1), lambda i: (i, 0)),
                  pl.BlockSpec((RB, H), lambda i: (0, 0)),
                  pl.BlockSpec((H, H), lambda i: (0, 0)),
                  pl.BlockSpec((H, C), lambda i: (0, 0))],
        out_specs=pl.BlockSpec((_BLK_E, C), lambda i: (i, 0)),
        out_shape=jax.ShapeDtypeStruct((E_PAD, C), f32),
    )(rb_p, sph_p, W_mlp1, W_mlp2, W_mlp3)

    partials = _agg_kernel(h, s, src, dst, zeros)

    W_sc_t = jnp.transpose(W_sc, (1, 0, 2))

    out = pl.pallas_call(
        _post_body,
        grid=(N // _BLK_N,),
        in_specs=[pl.BlockSpec((NC, _BLK_N, C), lambda i: (0, i, 0)),
                  pl.BlockSpec((_BLK_N, C), lambda i: (i, 0)),
                  pl.BlockSpec((_BLK_N, A), lambda i: (i, 0)),
                  pl.BlockSpec((C, C), lambda i: (0, 0)),
                  pl.BlockSpec((CORR, A, C), lambda i: (0, 0, 0)),
                  pl.BlockSpec((A, C, C), lambda i: (0, 0, 0)),
                  pl.BlockSpec((C, C), lambda i: (0, 0))],
        out_specs=pl.BlockSpec((_BLK_N, C), lambda i: (i, 0)),
        out_shape=jax.ShapeDtypeStruct((N, C), f32),
    )(partials[:, :N, :], node_features, node_attributes,
      W_post_int, W_contr, W_sc_t, W_post)

    return out


# speedup vs baseline: 1.9748x; 1.1513x over previous
import functools

import jax
import jax.numpy as jnp
from jax import lax
from jax.experimental import pallas as pl
from jax.experimental.pallas import tpu as pltpu
from jax.experimental.pallas import tpu_sc as plsc

N = 10000
E = 320000
C = 128
A = 10
RB = 8
H = 64
CORR = 3

NC = 2
NS = 16
NW = NC * NS
K = 128
NCHUNK = 79
EPW = K * NCHUNK
E_PAD = EPW * NW
ROWS_PER_TILE = 632
N_PAD = ROWS_PER_TILE * NS

_BLK_N = 1000
_BLK_E = 2048


def _pre_body(nf_ref, w_ref, h_ref):
    h_ref[...] = jnp.dot(nf_ref[...], w_ref[...],
                         preferred_element_type=jnp.float32)


def _scale_body(rb_ref, sph_ref, w1_ref, w2_ref, w3_ref, s_ref):
    x = jax.nn.silu(jnp.dot(rb_ref[...], w1_ref[...],
                            preferred_element_type=jnp.float32))
    x = jax.nn.silu(jnp.dot(x, w2_ref[...],
                            preferred_element_type=jnp.float32))
    w = jnp.dot(x, w3_ref[...], preferred_element_type=jnp.float32)
    s_ref[...] = w * sph_ref[...]


def _post_body(p_ref, nf_ref, attr_ref, wpi_ref, wc_ref, wsct_ref, wp_ref,
               out_ref):
    tm = p_ref[0] + p_ref[1]
    m = jnp.dot(tm, wpi_ref[...], preferred_element_type=jnp.float32)
    attr = attr_ref[...]
    m2 = m * m
    contracted = (jnp.dot(attr, wc_ref[0], preferred_element_type=jnp.float32) * m
                  + jnp.dot(attr, wc_ref[1], preferred_element_type=jnp.float32) * m2
                  + jnp.dot(attr, wc_ref[2], preferred_element_type=jnp.float32) * (m2 * m))
    nf = nf_ref[...]
    sc = jnp.zeros_like(m)
    for a in range(A):
        sc = sc + attr[:, a:a + 1] * jnp.dot(nf, wsct_ref[a],
                                             preferred_element_type=jnp.float32)
    out_ref[...] = jnp.dot(contracted + sc, wp_ref[...],
                           preferred_element_type=jnp.float32)


def _agg_body(h_hbm, s_hbm, src_hbm, dst_hbm, zeros_hbm, out_hbm,
              src_v, dst_v, rows_v, s_v, acc, sem):
    cid = lax.axis_index("c")
    sid = lax.axis_index("s")
    wid = sid * NC + cid
    base = wid * EPW
    my_rows = pl.ds(sid * ROWS_PER_TILE, ROWS_PER_TILE)

    pltpu.sync_copy(zeros_hbm.at[my_rows], acc.at[my_rows])
    plsc.subcore_barrier()

    def mul_row(r, carry):
        for gi in range(C // 16):
            sl = pl.ds(gi * 16, 16)
            rows_v[r, sl] = rows_v[r, sl] * s_v[r, sl]
        return carry

    def chunk_body(i, carry):
        off = base + i * K
        pltpu.sync_copy(src_hbm.at[pl.ds(off, K)], src_v)
        pltpu.sync_copy(dst_hbm.at[pl.ds(off, K)], dst_v)
        pltpu.async_copy(h_hbm.at[src_v], rows_v, sem).wait()
        pltpu.sync_copy(s_hbm.at[pl.ds(off, K)], s_v)
        lax.fori_loop(0, K, mul_row, 0)
        pltpu.sync_copy(rows_v, acc.at[dst_v], add=True)
        return carry

    lax.fori_loop(0, NCHUNK, chunk_body, 0)

    plsc.subcore_barrier()
    pltpu.sync_copy(acc.at[my_rows], out_hbm.at[cid].at[my_rows])


_agg_kernel = functools.partial(
    pl.kernel,
    out_type=jax.ShapeDtypeStruct((NC, N_PAD, C), jnp.float32),
    mesh=plsc.VectorSubcoreMesh(core_axis_name="c", subcore_axis_name="s"),
    scratch_types=[
        pltpu.VMEM((K,), jnp.int32),
        pltpu.VMEM((K,), jnp.int32),
        pltpu.VMEM((K, C), jnp.float32),
        pltpu.VMEM((K, C), jnp.float32),
        pltpu.VMEM_SHARED((N_PAD, C), jnp.float32),
        pltpu.SemaphoreType.DMA,
    ],
)(_agg_body)


def kernel(node_features, node_attributes, sph_harmonics, radial_basis,
           edge_index, W_pre, W_mlp1, W_mlp2, W_mlp3, W_post_int,
           W_contr, W_sc, W_post):
    f32 = jnp.float32
    pad = E_PAD - E
    src = jnp.concatenate([edge_index[0], jnp.zeros((pad,), jnp.int32)])
    dst = jnp.concatenate([edge_index[1], jnp.full((pad,), N, jnp.int32)])
    zeros = jnp.zeros((N_PAD, C), f32)

    h = pl.pallas_call(
        _pre_body,
        grid=(N // _BLK_N,),
        in_specs=[pl.BlockSpec((_BLK_N, C), lambda i: (i, 0)),
                  pl.BlockSpec((C, C), lambda i: (0, 0))],
        out_specs=pl.BlockSpec((_BLK_N, C), lambda i: (i, 0)),
        out_shape=jax.ShapeDtypeStruct((N, C), f32),
    )(node_features, W_pre)

    _last = E // _BLK_E
    s = pl.pallas_call(
        _scale_body,
        grid=(E_PAD // _BLK_E,),
        in_specs=[pl.BlockSpec((_BLK_E, RB), lambda i: (jnp.minimum(i, _last), 0)),
                  pl.BlockSpec((_BLK_E, 1), lambda i: (jnp.minimum(i, _last), 0)),
                  pl.BlockSpec((RB, H), lambda i: (0, 0)),
                  pl.BlockSpec((H, H), lambda i: (0, 0)),
                  pl.BlockSpec((H, C), lambda i: (0, 0))],
        out_specs=pl.BlockSpec((_BLK_E, C), lambda i: (i, 0)),
        out_shape=jax.ShapeDtypeStruct((E_PAD, C), f32),
    )(radial_basis, sph_harmonics, W_mlp1, W_mlp2, W_mlp3)

    partials = _agg_kernel(h, s, src, dst, zeros)

    W_sc_t = jnp.transpose(W_sc, (1, 0, 2))

    out = pl.pallas_call(
        _post_body,
        grid=(N // _BLK_N,),
        in_specs=[pl.BlockSpec((NC, _BLK_N, C), lambda i: (0, i, 0)),
                  pl.BlockSpec((_BLK_N, C), lambda i: (i, 0)),
                  pl.BlockSpec((_BLK_N, A), lambda i: (i, 0)),
                  pl.BlockSpec((C, C), lambda i: (0, 0)),
                  pl.BlockSpec((CORR, A, C), lambda i: (0, 0, 0)),
                  pl.BlockSpec((A, C, C), lambda i: (0, 0, 0)),
                  pl.BlockSpec((C, C), lambda i: (0, 0))],
        out_specs=pl.BlockSpec((_BLK_N, C), lambda i: (i, 0)),
        out_shape=jax.ShapeDtypeStruct((N, C), f32),
    )(partials[:, :N, :], node_features, node_attributes,
      W_post_int, W_contr, W_sc_t, W_post)

    return out
